# baseline (speedup 1.0000x reference)
import functools

import jax
import jax.numpy as jnp
from jax import lax
from jax.experimental import pallas as pl

H = W = 384
HW = H * W
CIN, NC = 96, 19
C0 = CIN + NC
O0 = C0 // 2
O1 = O0 // 2
GATE = 0.4
NEG = -3.0e38

BH = 16
NB = H // BH
PB = NB + 2
BL = BH * W
PHW = PB * BL


def _mask_body(c_ref, m_ref):
    c = c_ref[...]
    unc = 1.0 - jnp.max(c, axis=0)
    pad_r = jnp.full((1, W), NEG, jnp.float32)
    up = jnp.concatenate([unc[1:, :], pad_r], axis=0)
    dn = jnp.concatenate([pad_r, unc[:-1, :]], axis=0)
    v = jnp.maximum(jnp.maximum(unc, up), dn)
    pad_c = jnp.full((H, 1), NEG, jnp.float32)
    lf = jnp.concatenate([v[:, 1:], pad_c], axis=1)
    rt = jnp.concatenate([pad_c, v[:, :-1]], axis=1)
    d = jnp.maximum(jnp.maximum(v, lf), rt)
    m_ref[...] = jnp.where(d > GATE, 1.0, 0.0)


def _compute_mask(coarse):
    return pl.pallas_call(
        _mask_body,
        out_shape=jax.ShapeDtypeStruct((H, W), jnp.float32),
    )(coarse)


def _mlp_chunk(xc, ws):
    win, bin_, wm0, bm0, wm1, bm1, wout, bout = ws
    h = jnp.clip(jnp.dot(win, xc, preferred_element_type=jnp.float32) + bin_, 0.0, 6.0)
    h = h + jnp.clip(jnp.dot(wm0, h, preferred_element_type=jnp.float32) + bm0, 0.0, 6.0)
    h = h + jnp.clip(jnp.dot(wm1, h, preferred_element_type=jnp.float32) + bm1, 0.0, 6.0)
    return jnp.clip(jnp.dot(wout, h, preferred_element_type=jnp.float32) + bout, 0.0, 6.0)


def _mlp_body(n_in, m_ref, *refs):
    in_refs = refs[:n_in]
    w_refs = refs[n_in:n_in + 8]
    out_ref = refs[n_in + 8]
    ws = tuple(r[...] for r in w_refs)
    i = pl.program_id(0)
    interior = jnp.logical_and(i > 0, i < PB - 1)

    for row in range(BH):
        sl = slice(row * W, (row + 1) * W)
        parts = [r[:, sl] for r in in_refs]
        xc = parts[0] if n_in == 1 else jnp.concatenate(parts, axis=0)
        mrow = m_ref[:, row, :]
        act = jnp.logical_and(interior, jnp.max(mrow) > 0.5)

        @pl.when(act)
        def _(xc=xc, mrow=mrow, sl=sl):
            ur = _mlp_chunk(xc, ws)
            out_ref[:, sl] = jnp.where(mrow > 0.5, ur, xc)

        @pl.when(jnp.logical_not(act))
        def _(xc=xc, sl=sl):
            out_ref[:, sl] = jnp.where(interior, xc, 0.0)


def _run_mlp(in_arrays, in_padded, mask, wlist, cout):
    n_in = len(in_arrays)
    if in_padded:
        in_specs = [pl.BlockSpec((a.shape[0], BL), lambda i: (0, i))
                    for a in in_arrays]
    else:
        in_specs = [pl.BlockSpec((a.shape[0], BL),
                                 lambda i: (0, jnp.clip(i - 1, 0, NB - 1)))
                    for a in in_arrays]
    mask3 = mask.reshape(NB, BH, W)
    m_spec = pl.BlockSpec((1, BH, W), lambda i: (jnp.clip(i - 1, 0, NB - 1), 0, 0))
    w_specs = [pl.BlockSpec(w.shape, lambda i: (0, 0)) for w in wlist]
    return pl.pallas_call(
        functools.partial(_mlp_body, n_in),
        grid=(PB,),
        in_specs=[m_spec] + in_specs + w_specs,
        out_specs=pl.BlockSpec((cout, BL), lambda i: (0, i)),
        out_shape=jax.ShapeDtypeStruct((cout, PHW), jnp.float32),
    )(mask3, *in_arrays, *wlist)


def _sep_flat(v, dw, pw, be):
    Cc, Lv = v.shape
    Lo = Lv - 2 * W
    z = jnp.zeros((Cc, 1), jnp.float32)
    lane = lax.broadcasted_iota(jnp.int32, (1, Lv), 1)
    vm = jnp.where(lane % W == 0, 0.0,
                   jnp.concatenate([z, v[:, :-1]], axis=1))
    vp = jnp.where(lane % W == W - 1, 0.0,
                   jnp.concatenate([v[:, 1:], z], axis=1))
    acc = None
    for dh in range(3):
        o = dh * W
        t = (vm[:, o:o + Lo] * dw[:, dh, 0:1]
             + v[:, o:o + Lo] * dw[:, dh, 1:2]
             + vp[:, o:o + Lo] * dw[:, dh, 2:3])
        acc = t if acc is None else acc + t
    y = jnp.dot(pw, acc, preferred_element_type=jnp.float32) + be
    return jnp.maximum(y, 0.0)


def _conv_body(xp_ref, xc_ref, xn_ref,
               dw0_ref, pw0_ref, be0_ref,
               dw1_ref, pw1_ref, be1_ref,
               dwo_ref, pwo_ref, beo_ref, out_ref):
    i = pl.program_id(0)
    xs = jnp.concatenate(
        [xp_ref[:, 13 * W:], xc_ref[...], xn_ref[:, :3 * W]], axis=1)
    prow = lax.broadcasted_iota(jnp.int32, (1, 22 * W), 1) // W + (i * BH + 13)
    vb = jnp.where(
        jnp.logical_and(prow >= BH, prow < BH * (NB + 1)), 1.0, 0.0)
    xs = xs * vb
    t1 = xs[:, W:21 * W] + _sep_flat(xs, dw0_ref[...], pw0_ref[...], be0_ref[...])
    t1 = t1 * vb[:, W:21 * W]
    t2 = t1[:, W:19 * W] + _sep_flat(t1, dw1_ref[...], pw1_ref[...], be1_ref[...])
    t2 = t2 * vb[:, 2 * W:20 * W]
    out_ref[...] = _sep_flat(t2, dwo_ref[...], pwo_ref[...], beo_ref[...])


def _run_convs(xp, cw, cout, out_padded):
    C = xp.shape[0]
    xspec = lambda f: pl.BlockSpec((C, BL), f)
    in_specs = [xspec(lambda i: (0, i)),
                xspec(lambda i: (0, i + 1)),
                xspec(lambda i: (0, i + 2))]
    for w in cw:
        in_specs.append(pl.BlockSpec(w.shape, lambda i, n=w.ndim: (0,) * n))
    if out_padded:
        out_spec = pl.BlockSpec((cout, BL), lambda i: (0, i + 1))
        out_shape = jax.ShapeDtypeStruct((cout, PHW), jnp.float32)
    else:
        out_spec = pl.BlockSpec((cout, BL), lambda i: (0, i))
        out_shape = jax.ShapeDtypeStruct((cout, HW), jnp.float32)
    return pl.pallas_call(
        _conv_body,
        grid=(NB,),
        in_specs=in_specs,
        out_specs=out_spec,
        out_shape=out_shape,
    )(xp, xp, xp, *cw)


def _block_weights(p, b):
    wlist = [p[f'b{b}_win'], p[f'b{b}_bin'].reshape(-1, 1),
             p[f'b{b}_wm0'], p[f'b{b}_bm0'].reshape(-1, 1),
             p[f'b{b}_wm1'], p[f'b{b}_bm1'].reshape(-1, 1),
             p[f'b{b}_wout'], p[f'b{b}_bout'].reshape(-1, 1)]
    scale = 1.0 / jnp.sqrt(1.0 + 1e-5)
    cw = []
    for tag in ('0', '1', 'o'):
        dw = p[f'b{b}_dw{tag}'][:, 0]
        pw = p[f'b{b}_pw{tag}'][:, :, 0, 0]
        g = p[f'b{b}_g{tag}'] * scale
        pw_eff = pw * g[:, None]
        be = p[f'b{b}_be{tag}'].reshape(-1, 1)
        cw += [dw, pw_eff, be]
    return wlist, cw


def kernel(feature_map, coarse_pred, params):
    fm = feature_map[0].reshape(CIN, HW)
    cp0 = coarse_pred[0]
    cp = cp0.reshape(NC, HW)
    mask = _compute_mask(cp0)

    w0, c0 = _block_weights(params, 0)
    w1, c1 = _block_weights(params, 1)

    x0 = _run_mlp([fm, cp], False, mask, w0, C0)
    y0 = _run_convs(x0, c0, O0, out_padded=True)
    x1 = _run_mlp([y0], True, mask, w1, O0)
    y1 = _run_convs(x1, c1, O1, out_padded=False)
    return y1.reshape(1, O1, H, W)


# speedup vs baseline: 1.3197x; 1.3197x over previous
import functools

import jax
import jax.numpy as jnp
from jax import lax
from jax.experimental import pallas as pl

H = W = 384
HW = H * W
CIN, NC = 96, 19
C0 = CIN + NC
O0 = C0 // 2
O1 = O0 // 2
GATE = 0.4
NEG = -3.0e38

BH = 16
NB = H // BH
PB = NB + 2
BL = BH * W
PHW = PB * BL


def _mask_body(c_ref, m_ref):
    c = c_ref[...]
    unc = 1.0 - jnp.max(c, axis=0)
    pad_r = jnp.full((1, W), NEG, jnp.float32)
    up = jnp.concatenate([unc[1:, :], pad_r], axis=0)
    dn = jnp.concatenate([pad_r, unc[:-1, :]], axis=0)
    v = jnp.maximum(jnp.maximum(unc, up), dn)
    pad_c = jnp.full((H, 1), NEG, jnp.float32)
    lf = jnp.concatenate([v[:, 1:], pad_c], axis=1)
    rt = jnp.concatenate([pad_c, v[:, :-1]], axis=1)
    d = jnp.maximum(jnp.maximum(v, lf), rt)
    m_ref[...] = jnp.where(d > GATE, 1.0, 0.0)


def _compute_mask(coarse):
    return pl.pallas_call(
        _mask_body,
        out_shape=jax.ShapeDtypeStruct((H, W), jnp.float32),
    )(coarse)


def _mlp_chunk(xc, ws):
    win, bin_, wm0, bm0, wm1, bm1, wout, bout = ws
    h = jnp.clip(jnp.dot(win, xc, preferred_element_type=jnp.float32) + bin_, 0.0, 6.0)
    h = h + jnp.clip(jnp.dot(wm0, h, preferred_element_type=jnp.float32) + bm0, 0.0, 6.0)
    h = h + jnp.clip(jnp.dot(wm1, h, preferred_element_type=jnp.float32) + bm1, 0.0, 6.0)
    return jnp.clip(jnp.dot(wout, h, preferred_element_type=jnp.float32) + bout, 0.0, 6.0)


def _mlp_body(n_in, m_ref, *refs):
    in_refs = refs[:n_in]
    w_refs = refs[n_in:n_in + 8]
    out_ref = refs[n_in + 8]
    ws = tuple(r[...] for r in w_refs)
    i = pl.program_id(0)
    interior = jnp.logical_and(i > 0, i < PB - 1)

    for row in range(BH):
        sl = slice(row * W, (row + 1) * W)
        parts = [r[:, sl] for r in in_refs]
        xc = parts[0] if n_in == 1 else jnp.concatenate(parts, axis=0)
        mrow = m_ref[:, row, :]
        act = jnp.logical_and(interior, jnp.max(mrow) > 0.5)

        @pl.when(act)
        def _(xc=xc, mrow=mrow, sl=sl):
            ur = _mlp_chunk(xc.astype(jnp.float32), ws)
            out_ref[:, sl] = jnp.where(
                mrow > 0.5, ur, xc.astype(jnp.float32)).astype(out_ref.dtype)

        @pl.when(jnp.logical_not(act))
        def _(xc=xc, sl=sl):
            out_ref[:, sl] = jnp.where(
                interior, xc, 0).astype(out_ref.dtype)


def _run_mlp(in_arrays, in_padded, mask, wlist, cout):
    n_in = len(in_arrays)
    if in_padded:
        in_specs = [pl.BlockSpec((a.shape[0], BL), lambda i: (0, i))
                    for a in in_arrays]
    else:
        in_specs = [pl.BlockSpec((a.shape[0], BL),
                                 lambda i: (0, jnp.clip(i - 1, 0, NB - 1)))
                    for a in in_arrays]
    mask3 = mask.reshape(NB, BH, W)
    m_spec = pl.BlockSpec((1, BH, W), lambda i: (jnp.clip(i - 1, 0, NB - 1), 0, 0))
    w_specs = [pl.BlockSpec(w.shape, lambda i: (0, 0)) for w in wlist]
    return pl.pallas_call(
        functools.partial(_mlp_body, n_in),
        grid=(PB,),
        in_specs=[m_spec] + in_specs + w_specs,
        out_specs=pl.BlockSpec((cout, BL), lambda i: (0, i)),
        out_shape=jax.ShapeDtypeStruct((cout, PHW), jnp.bfloat16),
    )(mask3, *in_arrays, *wlist)


def _sep_flat(v, dw, pw, be):
    Cc, Lv = v.shape
    Lo = Lv - 2 * W
    z = jnp.zeros((Cc, 1), jnp.bfloat16)
    lane = lax.broadcasted_iota(jnp.int32, (1, Lv), 1)
    bm0 = jnp.where(lane % W == 0, 0.0, 1.0).astype(jnp.bfloat16)
    bm1 = jnp.where(lane % W == W - 1, 0.0, 1.0).astype(jnp.bfloat16)
    vm = jnp.concatenate([z, v[:, :-1]], axis=1) * bm0
    vp = jnp.concatenate([v[:, 1:], z], axis=1) * bm1
    acc = None
    for dh in range(3):
        o = dh * W
        t = (vm[:, o:o + Lo] * dw[:, dh, 0:1]
             + v[:, o:o + Lo] * dw[:, dh, 1:2]
             + vp[:, o:o + Lo] * dw[:, dh, 2:3])
        acc = t if acc is None else acc + t
    y = jnp.dot(pw, acc, preferred_element_type=jnp.float32) + be
    return jnp.maximum(y, 0.0)


def _conv_body(xp_ref, xc_ref, xn_ref,
               dw0_ref, pw0_ref, be0_ref,
               dw1_ref, pw1_ref, be1_ref,
               dwo_ref, pwo_ref, beo_ref, out_ref):
    i = pl.program_id(0)
    edge = jnp.logical_or(i == 0, i == NB - 1)
    xs = jnp.concatenate(
        [xp_ref[:, 13 * W:], xc_ref[...], xn_ref[:, :3 * W]], axis=1)
    prow = lax.broadcasted_iota(jnp.int32, (1, 22 * W), 1) // W + (i * BH + 13)
    vb = jnp.where(
        jnp.logical_and(prow >= BH, prow < BH * (NB + 1)),
        1.0, 0.0).astype(jnp.bfloat16)

    def zeropad(t, sl):
        return lax.cond(edge, lambda a: a * vb[:, sl], lambda a: a, t)

    y1 = _sep_flat(xs, dw0_ref[...], pw0_ref[...], be0_ref[...])
    t1 = xs[:, W:21 * W] + y1.astype(jnp.bfloat16)
    t1 = zeropad(t1, slice(W, 21 * W))
    y2 = _sep_flat(t1, dw1_ref[...], pw1_ref[...], be1_ref[...])
    t2 = t1[:, W:19 * W] + y2.astype(jnp.bfloat16)
    t2 = zeropad(t2, slice(2 * W, 20 * W))
    yo = _sep_flat(t2, dwo_ref[...], pwo_ref[...], beo_ref[...])
    out_ref[...] = yo.astype(out_ref.dtype)


def _run_convs(xp, cw, cout, out_padded):
    C = xp.shape[0]
    xspec = lambda f: pl.BlockSpec((C, BL), f)
    in_specs = [xspec(lambda i: (0, i)),
                xspec(lambda i: (0, i + 1)),
                xspec(lambda i: (0, i + 2))]
    for w in cw:
        in_specs.append(pl.BlockSpec(w.shape, lambda i, n=w.ndim: (0,) * n))
    if out_padded:
        out_spec = pl.BlockSpec((cout, BL), lambda i: (0, i + 1))
        out_shape = jax.ShapeDtypeStruct((cout, PHW), jnp.bfloat16)
    else:
        out_spec = pl.BlockSpec((cout, BL), lambda i: (0, i))
        out_shape = jax.ShapeDtypeStruct((cout, HW), jnp.float32)
    return pl.pallas_call(
        _conv_body,
        grid=(NB,),
        in_specs=in_specs,
        out_specs=out_spec,
        out_shape=out_shape,
    )(xp, xp, xp, *cw)


def _block_weights(p, b):
    wlist = [p[f'b{b}_win'], p[f'b{b}_bin'].reshape(-1, 1),
             p[f'b{b}_wm0'], p[f'b{b}_bm0'].reshape(-1, 1),
             p[f'b{b}_wm1'], p[f'b{b}_bm1'].reshape(-1, 1),
             p[f'b{b}_wout'], p[f'b{b}_bout'].reshape(-1, 1)]
    scale = 1.0 / jnp.sqrt(1.0 + 1e-5)
    cw = []
    for tag in ('0', '1', 'o'):
        dw = p[f'b{b}_dw{tag}'][:, 0].astype(jnp.bfloat16)
        pw = p[f'b{b}_pw{tag}'][:, :, 0, 0]
        g = p[f'b{b}_g{tag}'] * scale
        pw_eff = (pw * g[:, None]).astype(jnp.bfloat16)
        be = p[f'b{b}_be{tag}'].reshape(-1, 1)
        cw += [dw, pw_eff, be]
    return wlist, cw


def kernel(feature_map, coarse_pred, params):
    fm = feature_map[0].reshape(CIN, HW)
    cp0 = coarse_pred[0]
    cp = cp0.reshape(NC, HW)
    mask = _compute_mask(cp0)

    w0, c0 = _block_weights(params, 0)
    w1, c1 = _block_weights(params, 1)

    x0 = _run_mlp([fm, cp], False, mask, w0, C0)
    y0 = _run_convs(x0, c0, O0, out_padded=True)
    x1 = _run_mlp([y0], True, mask, w1, O0)
    y1 = _run_convs(x1, c1, O1, out_padded=False)
    return y1.reshape(1, O1, H, W)
